# Initial kernel scaffold; baseline (speedup 1.0000x reference)
import jax
import jax.numpy as jnp
from jax.experimental import pallas as pl


def kernel(indices, weight, A, B):
    raise NotImplementedError("write your pallas kernel here")


# speedup vs baseline: 20.6191x; 20.6191x over previous
import functools

import jax
import jax.numpy as jnp
from jax import lax
from jax.experimental import pallas as pl
from jax.experimental.pallas import tpu as pltpu
from jax.experimental.pallas import tpu_sc as plsc


def _fuse_body(a_ref, w_ref, b_ref, o_ref):
    o_ref[...] = w_ref[...] + jnp.dot(
        a_ref[...], b_ref[...], preferred_element_type=jnp.float32
    )


def _fuse_table(weight, A, B, rows_per_block=1000):
    vocab, dim = weight.shape
    rank = A.shape[1]
    grid = pl.cdiv(vocab, rows_per_block)
    return pl.pallas_call(
        _fuse_body,
        grid=(grid,),
        in_specs=[
            pl.BlockSpec((rows_per_block, rank), lambda i: (i, 0)),
            pl.BlockSpec((rows_per_block, dim), lambda i: (i, 0)),
            pl.BlockSpec((rank, dim), lambda i: (0, 0)),
        ],
        out_specs=pl.BlockSpec((rows_per_block, dim), lambda i: (i, 0)),
        out_shape=jax.ShapeDtypeStruct((vocab, dim), jnp.float32),
    )(A, weight, B)


def _sc_gather(table, idx_flat, chunk, n_chunks):
    n_tokens = idx_flat.shape[0]
    dim = table.shape[1]
    info = plsc.get_sparse_core_info()
    nc, ns = info.num_cores, info.num_subcores
    nw = nc * ns
    b_per_w = n_tokens // nw
    mesh = plsc.VectorSubcoreMesh(core_axis_name="c", subcore_axis_name="s")

    @functools.partial(
        pl.kernel,
        mesh=mesh,
        compiler_params=pltpu.CompilerParams(use_tc_tiling_on_sc=False),
        out_type=jax.ShapeDtypeStruct((n_tokens, dim), jnp.float32),
        scratch_types=[
            pltpu.VMEM((chunk,), jnp.int32),
            pltpu.VMEM((chunk, dim), jnp.float32),
            pltpu.SemaphoreType.DMA,
        ],
    )
    def gather_kernel(table_hbm, idx_hbm, out_hbm, idx_v, rows_v, sem):
        wid = lax.axis_index("s") * nc + lax.axis_index("c")
        base = wid * b_per_w

        def body(i, carry):
            off = base + i * chunk
            pltpu.sync_copy(idx_hbm.at[pl.ds(off, chunk)], idx_v)
            pltpu.async_copy(table_hbm.at[idx_v], rows_v, sem).wait()
            pltpu.sync_copy(rows_v, out_hbm.at[pl.ds(off, chunk)])
            return carry

        lax.fori_loop(0, n_chunks, body, 0)

    return gather_kernel(table, idx_flat)


def kernel(indices, weight, A, B):
    batch, hist = indices.shape
    n_tokens = batch * hist
    dim = weight.shape[1]
    idx_flat = indices.reshape(n_tokens).astype(jnp.int32)

    fused = _fuse_table(weight, A, B)

    nw = 32
    b_per_w = n_tokens // nw
    chunk = 1024
    while b_per_w % chunk != 0:
        chunk //= 2
    n_chunks = b_per_w // chunk

    flat = _sc_gather(fused, idx_flat, chunk, n_chunks)
    return flat.reshape(batch, hist, dim)


# speedup vs baseline: 20.7200x; 1.0049x over previous
import functools

import jax
import jax.numpy as jnp
from jax import lax
from jax.experimental import pallas as pl
from jax.experimental.pallas import tpu as pltpu
from jax.experimental.pallas import tpu_sc as plsc


def _fuse_body(a_ref, w_ref, b_ref, o_ref):
    o_ref[...] = w_ref[...] + jnp.dot(
        a_ref[...], b_ref[...], preferred_element_type=jnp.float32
    )


def _fuse_table(weight, A, B, rows_per_block=1000):
    vocab, dim = weight.shape
    rank = A.shape[1]
    grid = pl.cdiv(vocab, rows_per_block)
    return pl.pallas_call(
        _fuse_body,
        grid=(grid,),
        in_specs=[
            pl.BlockSpec((rows_per_block, rank), lambda i: (i, 0)),
            pl.BlockSpec((rows_per_block, dim), lambda i: (i, 0)),
            pl.BlockSpec((rank, dim), lambda i: (0, 0)),
        ],
        out_specs=pl.BlockSpec((rows_per_block, dim), lambda i: (i, 0)),
        out_shape=jax.ShapeDtypeStruct((vocab, dim), jnp.float32),
    )(A, weight, B)


def _sc_gather(table, idx2d, nb):
    batch, hist = idx2d.shape
    dim = table.shape[1]
    info = plsc.get_sparse_core_info()
    nc, ns = info.num_cores, info.num_subcores
    nw = nc * ns
    b_per_w = batch // nw
    n_chunks = b_per_w // nb
    mesh = plsc.VectorSubcoreMesh(core_axis_name="c", subcore_axis_name="s")

    @functools.partial(
        pl.kernel,
        mesh=mesh,
        compiler_params=pltpu.CompilerParams(use_tc_tiling_on_sc=False),
        out_type=jax.ShapeDtypeStruct((batch, hist, dim), jnp.float32),
        scratch_types=[
            pltpu.VMEM((b_per_w, hist), jnp.int32),
            pltpu.VMEM((nb, hist, dim), jnp.float32),
            pltpu.SemaphoreType.DMA,
            pltpu.SemaphoreType.DMA,
        ],
    )
    def gather_kernel(table_hbm, idx_hbm, out_hbm, idx_v, rows_v, gsem, ssem):
        wid = lax.axis_index("s") * nc + lax.axis_index("c")
        base = wid * b_per_w
        pltpu.sync_copy(idx_hbm.at[pl.ds(base, b_per_w)], idx_v)

        def body(i, carry):
            for j in range(nb):
                pltpu.async_copy(
                    table_hbm.at[idx_v.at[i * nb + j]], rows_v.at[j], gsem
                )
            for j in range(nb):
                pltpu.make_async_copy(
                    table_hbm.at[idx_v.at[i * nb + j]], rows_v.at[j], gsem
                ).wait()
            pltpu.async_copy(rows_v, out_hbm.at[pl.ds(base + i * nb, nb)], ssem)
            pltpu.make_async_copy(
                rows_v, out_hbm.at[pl.ds(base + i * nb, nb)], ssem
            ).wait()
            return carry

        lax.fori_loop(0, n_chunks, body, 0)

    return gather_kernel(table, idx2d)


def kernel(indices, weight, A, B):
    fused = _fuse_table(weight, A, B)
    return _sc_gather(fused, indices.astype(jnp.int32), nb=4)
